# baseline (speedup 1.0000x reference)
import functools

import jax
import jax.numpy as jnp
from jax import lax
from jax.experimental import pallas as pl
from jax.experimental.pallas import tpu as pltpu
from jax.experimental.pallas import tpu_sc as plsc

_ROWS = 1024
_COLS = 100000
_D = 32

_RB = 16
_NSTEP = _ROWS // _RB
_TB = 1568
_FLAT = _NSTEP * _TB * _D


def _argmax_body(x_ref, t_ref, idx_ref, flat_ref, idx_acc):
    i = pl.program_id(0)
    cols = lax.broadcasted_iota(jnp.int32, (_RB, _COLS), 1)
    vals = jnp.where(cols < _COLS, x_ref[...], -jnp.inf)
    bmax = jnp.max(vals, axis=1)
    arg = jnp.min(jnp.where(vals == bmax[:, None], cols, _COLS), axis=1)
    idx_acc[i, :] = arg
    t = t_ref[...]
    j2 = lax.broadcasted_iota(jnp.int32, (_TB // 4, _TB), 0)
    r2 = lax.broadcasted_iota(jnp.int32, (_TB // 4, _TB), 1)
    flat_ref[...] = jnp.concatenate(
        [
            jnp.dot((r2 == 4 * j2 + q).astype(jnp.float32), t,
                    precision=lax.Precision.HIGHEST,
                    preferred_element_type=jnp.float32)
            for q in range(4)
        ],
        axis=1,
    )

    @pl.when(i == _NSTEP - 1)
    def _out():
        idx_ref[...] = idx_acc[...]


_argmax_call = pl.pallas_call(
    _argmax_body,
    grid=(_NSTEP,),
    in_specs=[
        pl.BlockSpec((_RB, _COLS), lambda i: (i, 0)),
        pl.BlockSpec((_TB, _D), lambda i: (i, 0)),
    ],
    out_specs=[
        pl.BlockSpec((_NSTEP, _RB), lambda i: (0, 0)),
        pl.BlockSpec((_TB // 4, 4 * _D), lambda i: (i, 0)),
    ],
    out_shape=[
        jax.ShapeDtypeStruct((_NSTEP, _RB), jnp.int32),
        jax.ShapeDtypeStruct((_FLAT // (4 * _D), 4 * _D), jnp.float32),
    ],
    scratch_shapes=[pltpu.VMEM((_NSTEP, _RB), jnp.int32)],
)

_info = plsc.get_sparse_core_info()
_NW = _info.num_cores * _info.num_subcores
_EPW = _ROWS * _D // _NW
_CHUNK = 128
_NDMA = _EPW // _CHUNK


_BPW = _ROWS // _NW


def _gather_body(table_hbm, idx_hbm, out_hbm, idx_v, eidx_v, out_v, sem):
    wid = lax.axis_index("s") * _info.num_cores + lax.axis_index("c")
    pltpu.sync_copy(idx_hbm.at[pl.ds(wid * (_BPW // 16), _BPW // 16)], idx_v)

    lane = lax.iota(jnp.int32, 16)

    def _build(b, _):
        g = lax.div(b, 16)
        rows16 = idx_v[g]
        sel = jnp.full((16, 1), lax.rem(b, 16), jnp.int32)
        rowb = lax.gather(
            rows16, sel,
            lax.GatherDimensionNumbers(
                offset_dims=(), collapsed_slice_dims=(0,),
                start_index_map=(0,)),
            slice_sizes=(1,),
            mode=lax.GatherScatterMode.PROMISE_IN_BOUNDS)
        base = rowb * _D + lane
        eidx_v[pl.ds(b * _D, 16)] = base
        eidx_v[pl.ds(b * _D + 16, 16)] = base + 16
        return _

    lax.fori_loop(0, _BPW, _build, 0)

    cps = [
        pltpu.async_copy(
            table_hbm.at[eidx_v.at[pl.ds(j * _CHUNK, _CHUNK)]],
            out_v.at[pl.ds(j * _CHUNK, _CHUNK)],
            sem,
        )
        for j in range(_NDMA)
    ]
    for cp in cps:
        cp.wait()
    pltpu.sync_copy(out_v, out_hbm.at[pl.ds(wid * _EPW, _EPW)])


_gather_call = functools.partial(
    pl.kernel,
    mesh=plsc.VectorSubcoreMesh(core_axis_name="c", subcore_axis_name="s"),
    out_type=jax.ShapeDtypeStruct((_ROWS * _D,), jnp.float32),
    scratch_types=[
        pltpu.VMEM((_BPW // 16, 16), jnp.int32),
        pltpu.VMEM((_EPW,), jnp.int32),
        pltpu.VMEM((_EPW,), jnp.float32),
        pltpu.SemaphoreType.DMA,
    ],
)(_gather_body)


def kernel(x, table):
    idx, flat_table = _argmax_call(x, table)
    flat = _gather_call(flat_table.reshape(-1), idx)
    return flat.reshape(_ROWS, _D)


# speedup vs baseline: 1.6145x; 1.6145x over previous
import functools

import jax
import jax.numpy as jnp
from jax import lax
from jax.experimental import pallas as pl
from jax.experimental.pallas import tpu as pltpu
from jax.experimental.pallas import tpu_sc as plsc

_ROWS = 1024
_COLS = 100000
_D = 32

_RB = 16
_NSTEP = _ROWS // _RB
_TB = 1568
_FLAT = _NSTEP * _TB * _D


def _argmax_body(x_ref, t_ref, idx_ref, flat_ref, idx_acc):
    i = pl.program_id(0)
    cols = lax.broadcasted_iota(jnp.int32, (_RB, _COLS), 1)
    vals = jnp.where(cols < _COLS, x_ref[...], -jnp.inf)
    bmax = jnp.max(vals, axis=1)
    arg = jnp.min(jnp.where(vals == bmax[:, None], cols, _COLS), axis=1)
    idx_acc[i, :] = arg
    t = t_ref[...]
    j2 = lax.broadcasted_iota(jnp.int32, (_TB // 4, _TB), 0)
    r2 = lax.broadcasted_iota(jnp.int32, (_TB // 4, _TB), 1)
    sel = (r2 >> 2 == j2).astype(jnp.float32)
    rq = lax.broadcasted_iota(jnp.int32, (_TB, 1), 0) & 3
    t4 = jnp.concatenate(
        [jnp.where(rq == q, t, 0.0) for q in range(4)], axis=1)
    flat_ref[...] = jnp.dot(sel, t4,
                            precision=lax.Precision.HIGHEST,
                            preferred_element_type=jnp.float32)

    @pl.when(i == _NSTEP - 1)
    def _out():
        idx_ref[...] = idx_acc[...]


_argmax_call = pl.pallas_call(
    _argmax_body,
    grid=(_NSTEP,),
    in_specs=[
        pl.BlockSpec((_RB, _COLS), lambda i: (i, 0)),
        pl.BlockSpec((_TB, _D), lambda i: (i, 0)),
    ],
    out_specs=[
        pl.BlockSpec((_NSTEP, _RB), lambda i: (0, 0)),
        pl.BlockSpec((_TB // 4, 4 * _D), lambda i: (i, 0)),
    ],
    out_shape=[
        jax.ShapeDtypeStruct((_NSTEP, _RB), jnp.int32),
        jax.ShapeDtypeStruct((_FLAT // (4 * _D), 4 * _D), jnp.float32),
    ],
    scratch_shapes=[pltpu.VMEM((_NSTEP, _RB), jnp.int32)],
)

_info = plsc.get_sparse_core_info()
_NW = _info.num_cores * _info.num_subcores
_EPW = _ROWS * _D // _NW
_CHUNK = 128
_NDMA = _EPW // _CHUNK


_BPW = _ROWS // _NW


def _gather_body(table_hbm, idx_hbm, out_hbm, idx_v, eidx_v, out_v, sem):
    wid = lax.axis_index("s") * _info.num_cores + lax.axis_index("c")
    pltpu.sync_copy(idx_hbm.at[pl.ds(wid * (_BPW // 16), _BPW // 16)], idx_v)

    lane = lax.iota(jnp.int32, 16)

    def _build(b, _):
        g = lax.div(b, 16)
        rows16 = idx_v[g]
        sel = jnp.full((16, 1), lax.rem(b, 16), jnp.int32)
        rowb = lax.gather(
            rows16, sel,
            lax.GatherDimensionNumbers(
                offset_dims=(), collapsed_slice_dims=(0,),
                start_index_map=(0,)),
            slice_sizes=(1,),
            mode=lax.GatherScatterMode.PROMISE_IN_BOUNDS)
        base = rowb * _D + lane
        eidx_v[pl.ds(b * _D, 16)] = base
        eidx_v[pl.ds(b * _D + 16, 16)] = base + 16
        return _

    lax.fori_loop(0, _BPW, _build, 0)

    cps = [
        pltpu.async_copy(
            table_hbm.at[eidx_v.at[pl.ds(j * _CHUNK, _CHUNK)]],
            out_v.at[pl.ds(j * _CHUNK, _CHUNK)],
            sem,
        )
        for j in range(_NDMA)
    ]
    for cp in cps:
        cp.wait()
    pltpu.sync_copy(out_v, out_hbm.at[pl.ds(wid * _EPW, _EPW)])


_gather_call = functools.partial(
    pl.kernel,
    mesh=plsc.VectorSubcoreMesh(core_axis_name="c", subcore_axis_name="s"),
    out_type=jax.ShapeDtypeStruct((_ROWS * _D,), jnp.float32),
    scratch_types=[
        pltpu.VMEM((_BPW // 16, 16), jnp.int32),
        pltpu.VMEM((_EPW,), jnp.int32),
        pltpu.VMEM((_EPW,), jnp.float32),
        pltpu.SemaphoreType.DMA,
    ],
)(_gather_body)


def kernel(x, table):
    idx, flat_table = _argmax_call(x, table)
    flat = _gather_call(flat_table.reshape(-1), idx)
    return flat.reshape(_ROWS, _D)


# speedup vs baseline: 1.8588x; 1.1513x over previous
import functools

import jax
import jax.numpy as jnp
from jax import lax
from jax.experimental import pallas as pl
from jax.experimental.pallas import tpu as pltpu
from jax.experimental.pallas import tpu_sc as plsc

_ROWS = 1024
_COLS = 100000
_D = 32

_CB = 4096
_NSTEP = pl.cdiv(_COLS, _CB)
_TB = 4016
_TROWS = _NSTEP * _TB


def _argmax_body(x_ref, t_ref, idx_ref, flat_ref, max_s, arg_s):
    j = pl.program_id(0)

    @pl.when(j == 0)
    def _init():
        max_s[...] = jnp.full((_ROWS,), -jnp.inf, jnp.float32)
        arg_s[...] = jnp.zeros((_ROWS,), jnp.int32)

    cols = j * _CB + lax.broadcasted_iota(jnp.int32, (_ROWS, _CB), 1)
    vals = jnp.where(cols < _COLS, x_ref[...], -jnp.inf)
    bmax = jnp.max(vals, axis=1)
    barg = jnp.min(jnp.where(vals == bmax[:, None], cols, _COLS), axis=1)
    upd = bmax > max_s[...]
    arg_s[...] = jnp.where(upd, barg, arg_s[...])
    max_s[...] = jnp.where(upd, bmax, max_s[...])

    flat_ref[...] = jnp.concatenate(
        [t_ref[...], jnp.zeros((_TB, 3 * _D), jnp.float32)], axis=1)

    @pl.when(j == _NSTEP - 1)
    def _out():
        idx_ref[...] = arg_s[...]


_argmax_call = pl.pallas_call(
    _argmax_body,
    grid=(_NSTEP,),
    in_specs=[
        pl.BlockSpec((_ROWS, _CB), lambda j: (0, j)),
        pl.BlockSpec((_TB, _D), lambda j: (j, 0)),
    ],
    out_specs=[
        pl.BlockSpec((_ROWS,), lambda j: (0,)),
        pl.BlockSpec((_TB, 4 * _D), lambda j: (j, 0)),
    ],
    out_shape=[
        jax.ShapeDtypeStruct((_ROWS,), jnp.int32),
        jax.ShapeDtypeStruct((_TROWS, 4 * _D), jnp.float32),
    ],
    scratch_shapes=[
        pltpu.VMEM((_ROWS,), jnp.float32),
        pltpu.VMEM((_ROWS,), jnp.int32),
    ],
)

_info = plsc.get_sparse_core_info()
_NW = _info.num_cores * _info.num_subcores
_BPW = _ROWS // _NW


def _gather_body(table_hbm, idx_hbm, out_hbm, idx_v, rows_v, sem):
    wid = lax.axis_index("s") * _info.num_cores + lax.axis_index("c")
    base = wid * _BPW
    pltpu.sync_copy(idx_hbm.at[pl.ds(base, _BPW)], idx_v)
    pltpu.async_copy(table_hbm.at[idx_v], rows_v, sem).wait()
    pltpu.sync_copy(rows_v, out_hbm.at[pl.ds(base, _BPW)])


_gather_call = functools.partial(
    pl.kernel,
    mesh=plsc.VectorSubcoreMesh(core_axis_name="c", subcore_axis_name="s"),
    out_type=jax.ShapeDtypeStruct((_ROWS, 4 * _D), jnp.float32),
    scratch_types=[
        pltpu.VMEM((_BPW,), jnp.int32),
        pltpu.VMEM((_BPW, 4 * _D), jnp.float32),
        pltpu.SemaphoreType.DMA,
    ],
)(_gather_body)


def kernel(x, table):
    idx, flat4 = _argmax_call(x, table)
    out128 = _gather_call(flat4, idx)
    return out128[:, :_D]


# speedup vs baseline: 1.9043x; 1.0245x over previous
import functools

import jax
import jax.numpy as jnp
from jax import lax
from jax.experimental import pallas as pl
from jax.experimental.pallas import tpu as pltpu
from jax.experimental.pallas import tpu_sc as plsc

_ROWS = 1024
_COLS = 100000
_D = 32

_CB = 3840
_NBUF = 3
_NFULL = _COLS // _CB
_NITER = 8
_TAIL = _COLS - _CB * _NFULL

_info = plsc.get_sparse_core_info()
_NW = _info.num_cores * _info.num_subcores
_BPW = _ROWS // _NW
_SPW = 3120
_REM = _COLS - _SPW * _NW


def _wid():
    return lax.axis_index("s") * _info.num_cores + lax.axis_index("c")


_CH = 120
_NCH = _SPW // _CH


def _widen(src, dst, n):
    def body(r, carry):
        dst[r, pl.ds(0, 16)] = src[r, pl.ds(0, 16)]
        dst[r, pl.ds(16, 16)] = src[r, pl.ds(16, 16)]
        return carry

    lax.fori_loop(0, n, body, 0)


def _stage_body(t_hbm, flat4_hbm, slab_v, wide_v, rem_v, wrem_v):
    w = _wid()
    base = w * _SPW
    for c in range(_NCH):
        off = base + c * _CH
        pltpu.sync_copy(t_hbm.at[pl.ds(off, _CH)], slab_v)
        _widen(slab_v, wide_v, _CH)
        pltpu.sync_copy(wide_v, flat4_hbm.at[pl.ds(off, _CH)])

    @pl.when(w < _REM // 8)
    def _leftover():
        rbase = _SPW * _NW + w * 8
        pltpu.sync_copy(t_hbm.at[pl.ds(rbase, 8)], rem_v)
        _widen(rem_v, wrem_v, 8)
        pltpu.sync_copy(wrem_v, flat4_hbm.at[pl.ds(rbase, 8)])


_stage_call = functools.partial(
    pl.kernel,
    mesh=plsc.VectorSubcoreMesh(core_axis_name="c", subcore_axis_name="s"),
    out_type=jax.ShapeDtypeStruct((_COLS, 4 * _D), jnp.float32),
    scratch_types=[
        pltpu.VMEM((_CH, _D), jnp.float32),
        pltpu.VMEM((_CH, 4 * _D), jnp.float32),
        pltpu.VMEM((8, _D), jnp.float32),
        pltpu.VMEM((8, 4 * _D), jnp.float32),
    ],
)(_stage_body)


def _argmax_body(x_hbm, xt_ref, idx_ref, b0, b1, b2, max_s, arg_s, s0, s1, s2):
    bufs = (b0, b1, b2)
    sems = (s0, s1, s2)

    def xcp(j, k):
        off = pl.multiple_of(j * _CB, _CB)
        return pltpu.make_async_copy(
            x_hbm.at[:, pl.ds(off, _CB)], bufs[k], sems[k])

    for k in range(_NBUF):
        xcp(k, k).start()
    max_s[...] = jnp.full((_ROWS,), -jnp.inf, jnp.float32)
    arg_s[...] = jnp.zeros((_ROWS,), jnp.int32)

    def merge(vals, cols):
        bmax = jnp.max(vals, axis=1)
        barg = jnp.min(jnp.where(vals == bmax[:, None], cols, _COLS), axis=1)
        upd = bmax > max_s[...]
        arg_s[...] = jnp.where(upd, barg, arg_s[...])
        max_s[...] = jnp.where(upd, bmax, max_s[...])

    def body(jj, carry):
        for k in range(_NBUF):
            j = jj * _NBUF + k
            xcp(j, k).wait()
            cols = j * _CB + lax.broadcasted_iota(jnp.int32, (_ROWS, _CB), 1)
            merge(bufs[k][...], cols)

            @pl.when(j + _NBUF < _NFULL)
            def _prefetch():
                xcp(j + _NBUF, k).start()

        return carry

    lax.fori_loop(0, _NITER, body, 0)

    for k in range(_NFULL - _NITER * _NBUF):
        j = _NITER * _NBUF + k
        xcp(j, k).wait()
        cols = j * _CB + lax.broadcasted_iota(jnp.int32, (_ROWS, _CB), 1)
        merge(bufs[k][...], cols)

    tcols = _NFULL * _CB + lax.broadcasted_iota(jnp.int32, (_ROWS, _TAIL), 1)
    merge(jnp.where(tcols < _COLS, xt_ref[...], -jnp.inf), tcols)

    idx_ref[...] = arg_s[...]


_argmax_call = pl.pallas_call(
    _argmax_body,
    in_specs=[
        pl.BlockSpec(memory_space=pltpu.MemorySpace.HBM),
        pl.BlockSpec(memory_space=pltpu.MemorySpace.VMEM),
    ],
    out_specs=pl.BlockSpec(memory_space=pltpu.MemorySpace.VMEM),
    out_shape=jax.ShapeDtypeStruct((_ROWS,), jnp.int32),
    scratch_shapes=[
        pltpu.VMEM((_ROWS, _CB), jnp.float32),
        pltpu.VMEM((_ROWS, _CB), jnp.float32),
        pltpu.VMEM((_ROWS, _CB), jnp.float32),
        pltpu.VMEM((_ROWS,), jnp.float32),
        pltpu.VMEM((_ROWS,), jnp.int32),
        pltpu.SemaphoreType.DMA,
        pltpu.SemaphoreType.DMA,
        pltpu.SemaphoreType.DMA,
    ],
)


def _gather_body(table_hbm, idx_hbm, out_hbm, idx_v, rows_v, sem):
    base = _wid() * _BPW
    pltpu.sync_copy(idx_hbm.at[pl.ds(base, _BPW)], idx_v)
    pltpu.async_copy(table_hbm.at[idx_v], rows_v, sem).wait()
    pltpu.sync_copy(rows_v, out_hbm.at[pl.ds(base, _BPW)])


_gather_call = functools.partial(
    pl.kernel,
    mesh=plsc.VectorSubcoreMesh(core_axis_name="c", subcore_axis_name="s"),
    out_type=jax.ShapeDtypeStruct((_ROWS, 4 * _D), jnp.float32),
    scratch_types=[
        pltpu.VMEM((_BPW,), jnp.int32),
        pltpu.VMEM((_BPW, 4 * _D), jnp.float32),
        pltpu.SemaphoreType.DMA,
    ],
)(_gather_body)


def kernel(x, table):
    xt = lax.slice(x, (0, _NFULL * _CB), (_ROWS, _COLS))
    flat4 = _stage_call(table)
    idx = _argmax_call(x, xt)
    out128 = _gather_call(flat4, idx)
    return out128[:, :_D]
